# Initial kernel scaffold; baseline (speedup 1.0000x reference)
import jax
import jax.numpy as jnp
from jax import lax
from jax.experimental import pallas as pl
from jax.experimental.pallas import tpu as pltpu
from jax.experimental.pallas import tpu_sc as plsc

_B = 4096
_F = 26
_V = 100000
_D = 32
_NC = 16

_NW = 32
_BPW = _B // _NW
_CHUNK = 32
_NCHUNK = _BPW // _CHUNK
_F2 = 2 * _F
_ROWS = _F2 * _CHUNK
_SEG = 128
_NSEG = _ROWS // _SEG


def _sc_body(emb, cat2, num2, wdup, kdup, foff, out,
             catv, numv, wv, kv, foffv, idxv, rowsv, accv, outv, sem):
    wid = lax.axis_index("s") * 2 + lax.axis_index("c")
    b0 = wid * _BPW
    iota = lax.iota(jnp.int32, 16)
    zero16 = jnp.zeros((16,), jnp.float32)

    pltpu.sync_copy(cat2.at[pl.ds(b0, _BPW), :], catv)
    pltpu.sync_copy(num2.at[pl.ds(b0, _BPW), :], numv)
    pltpu.sync_copy(wdup, wv)
    pltpu.sync_copy(kdup, kv)
    pltpu.sync_copy(foff, foffv)

    for c in range(_NCHUNK):
        jbase = c * _CHUNK

        def build(fp, _):
            fpv = jnp.full((16,), fp, dtype=jnp.int32)
            fofb = plsc.load_gather(foffv, [fpv])
            for g in range(2):
                jvec = iota + (jbase + g * 16)
                catg = plsc.load_gather(catv, [jvec, fpv])
                r0 = fp * 32 + g * 16
                idxv[r0 // _SEG, pl.ds(lax.rem(r0, _SEG), 16)] = catg + fofb
            return 0

        lax.fori_loop(0, _F2, build, 0)

        descs = [
            pltpu.async_copy(emb.at[idxv.at[s]],
                             rowsv.at[pl.ds(s * _SEG, _SEG)], sem)
            for s in range(_NSEG)
        ]
        for d in descs:
            d.wait()

        def zero(j, _):
            accv[j, pl.ds(0, 16)] = zero16
            accv[j, pl.ds(16, 16)] = zero16
            return 0

        lax.fori_loop(0, _CHUNK, zero, 0)

        def field(fp, _):
            w0 = wv[fp, pl.ds(0, 16)]
            w1 = wv[fp, pl.ds(16, 16)]
            rbase = fp * _CHUNK

            def row(j, _):
                r = rbase + j
                plsc.addupdate(accv.at[j, pl.ds(0, 16)],
                               rowsv[r, pl.ds(0, 16)] * w0)
                plsc.addupdate(accv.at[j, pl.ds(16, 16)],
                               rowsv[r, pl.ds(16, 16)] * w1)
                return 0

            lax.fori_loop(0, _CHUNK, row, 0)
            return 0

        lax.fori_loop(0, _F2, field, 0)

        k0 = kv[pl.ds(0, 16)]
        k1 = kv[pl.ds(16, 16)]

        def numrow(j, _):
            jg = jbase + j
            plsc.addupdate(accv.at[j, pl.ds(0, 16)],
                           numv[jg, pl.ds(0, 16)] * k0)
            plsc.addupdate(accv.at[j, pl.ds(16, 16)],
                           numv[jg, pl.ds(16, 16)] * k1)
            return 0

        lax.fori_loop(0, _CHUNK, numrow, 0)

        for g in range(2):
            jvec = iota + g * 16

            def red(d, a):
                dv = jnp.full((16,), d, dtype=jnp.int32)
                return a + plsc.load_gather(accv, [jvec, dv])

            res = lax.fori_loop(0, _D, red, zero16)
            outv[pl.ds(jbase + g * 16, 16)] = res

    pltpu.sync_copy(outv, out.at[pl.ds(b0, _BPW)])


@jax.jit
def _run(emb_flat, cat2, num2, wdup, kdup, foff):
    mesh = plsc.VectorSubcoreMesh(core_axis_name="c", subcore_axis_name="s",
                                  num_cores=2, num_subcores=16)
    fn = pl.kernel(
        _sc_body,
        out_type=jax.ShapeDtypeStruct((_B,), jnp.float32),
        mesh=mesh,
        scratch_types=[
            pltpu.VMEM((_BPW, _F2), jnp.int32),
            pltpu.VMEM((_BPW, _D), jnp.float32),
            pltpu.VMEM((_F2, _D), jnp.float32),
            pltpu.VMEM((_D,), jnp.float32),
            pltpu.VMEM((_F2,), jnp.int32),
            pltpu.VMEM((_NSEG, _SEG), jnp.int32),
            pltpu.VMEM((_ROWS, _D), jnp.float32),
            pltpu.VMEM((_CHUNK, _D), jnp.float32),
            pltpu.VMEM((_BPW,), jnp.float32),
            pltpu.SemaphoreType.DMA,
        ],
    )
    return fn(emb_flat, cat2, num2, wdup, kdup, foff)


def kernel(pos_cat, pos_num, neg_cat, neg_num, emb_tables, lin_w, lin_b,
           bn_gamma, bn_beta, bn_mean, bn_var):
    emb_flat = emb_tables.reshape(_F * _V, _D)
    cat2 = jnp.concatenate([pos_cat, neg_cat], axis=1)
    num2 = jnp.concatenate([pos_num, neg_num], axis=1)
    w_emb = lin_w[0, : _F * _D].reshape(_F, _D)
    wdup = jnp.concatenate([w_emb, -w_emb], axis=0)
    knum = lin_w[0, _F * _D:] * bn_gamma * lax.rsqrt(bn_var + 1e-5)
    kdup = jnp.concatenate([knum, -knum], axis=0)
    foff = jnp.tile(jnp.arange(_F, dtype=jnp.int32) * _V, 2)
    out = _run(emb_flat, cat2, num2, wdup, kdup, foff)
    return out.reshape(_B, 1)


# speedup vs baseline: 4.1974x; 4.1974x over previous
import jax
import jax.numpy as jnp
from jax import lax
from jax.experimental import pallas as pl
from jax.experimental.pallas import tpu as pltpu
from jax.experimental.pallas import tpu_sc as plsc

_B = 4096
_F = 26
_V = 100000
_D = 32
_NC = 16

_NW = 32
_BPW = _B // _NW
_CHUNK = 32
_NCHUNK = _BPW // _CHUNK
_F2 = 2 * _F
_ROWS = _F2 * _CHUNK
_SEG = 128
_NSEG = _ROWS // _SEG


def _sc_body(emb, cat2, num2, wdup, kdup, foff, out,
             catv, numv, wv, kv, foffv, idxv, rowsv, accv, outv, sem):
    wid = lax.axis_index("s") * 2 + lax.axis_index("c")
    b0 = wid * _BPW
    iota = lax.iota(jnp.int32, 16)
    zero16 = jnp.zeros((16,), jnp.float32)

    pltpu.sync_copy(cat2.at[pl.ds(b0, _BPW), :], catv)
    pltpu.sync_copy(num2.at[pl.ds(b0, _BPW), :], numv)
    pltpu.sync_copy(wdup, wv)
    pltpu.sync_copy(kdup, kv)
    pltpu.sync_copy(foff, foffv)

    for c in range(_NCHUNK):
        jbase = c * _CHUNK

        def build(fp, _):
            fpv = jnp.full((16,), fp, dtype=jnp.int32)
            fofb = plsc.load_gather(foffv, [fpv])
            for g in range(2):
                jvec = iota + (jbase + g * 16)
                catg = plsc.load_gather(catv, [jvec, fpv])
                r0 = fp * 32 + g * 16
                idxv[r0 // _SEG, pl.ds(lax.rem(r0, _SEG), 16)] = catg + fofb
            return 0

        lax.fori_loop(0, _F2, build, 0)

        descs = [
            pltpu.async_copy(emb.at[idxv.at[s]],
                             rowsv.at[pl.ds(s * _SEG, _SEG)], sem)
            for s in range(_NSEG)
        ]
        for d in descs:
            d.wait()

        def zero(j, _):
            accv[j, pl.ds(0, 16)] = zero16
            accv[j, pl.ds(16, 16)] = zero16
            return 0

        lax.fori_loop(0, _CHUNK, zero, 0)

        def field(fp, _):
            w0 = wv[fp, pl.ds(0, 16)]
            w1 = wv[fp, pl.ds(16, 16)]
            rbase = fp * _CHUNK

            def row(j, _):
                r = rbase + j
                plsc.addupdate(accv.at[j, pl.ds(0, 16)],
                               rowsv[r, pl.ds(0, 16)] * w0)
                plsc.addupdate(accv.at[j, pl.ds(16, 16)],
                               rowsv[r, pl.ds(16, 16)] * w1)
                return 0

            lax.fori_loop(0, _CHUNK, row, 0)
            return 0

        lax.fori_loop(0, _F2, field, 0)

        k0 = kv[pl.ds(0, 16)]
        k1 = kv[pl.ds(16, 16)]

        def numrow(j, _):
            jg = jbase + j
            plsc.addupdate(accv.at[j, pl.ds(0, 16)],
                           numv[jg, pl.ds(0, 16)] * k0)
            plsc.addupdate(accv.at[j, pl.ds(16, 16)],
                           numv[jg, pl.ds(16, 16)] * k1)
            return 0

        lax.fori_loop(0, _CHUNK, numrow, 0)

        for g in range(2):
            jvec = iota + g * 16

            def red(d, a):
                dv = jnp.full((16,), d, dtype=jnp.int32)
                return a + plsc.load_gather(accv, [jvec, dv])

            res = lax.fori_loop(0, _D, red, zero16)
            outv[pl.ds(jbase + g * 16, 16)] = res

    pltpu.sync_copy(outv, out.at[pl.ds(b0, _BPW)])


@jax.jit
def _run(emb_flat, cat2, num2, wdup, kdup, foff):
    mesh = plsc.VectorSubcoreMesh(core_axis_name="c", subcore_axis_name="s",
                                  num_cores=2, num_subcores=16)
    fn = pl.kernel(
        _sc_body,
        out_type=jax.ShapeDtypeStruct((_B,), jnp.float32),
        mesh=mesh,
        scratch_types=[
            pltpu.VMEM((_BPW, _F2), jnp.int32),
            pltpu.VMEM((_BPW, _D), jnp.float32),
            pltpu.VMEM((_F2, _D), jnp.float32),
            pltpu.VMEM((_D,), jnp.float32),
            pltpu.VMEM((_F2,), jnp.int32),
            pltpu.VMEM((_NSEG, _SEG), jnp.int32),
            pltpu.VMEM((_ROWS, _D), jnp.float32),
            pltpu.VMEM((_CHUNK, _D), jnp.float32),
            pltpu.VMEM((_BPW,), jnp.float32),
            pltpu.SemaphoreType.DMA,
        ],
        compiler_params=pltpu.CompilerParams(needs_layout_passes=False,
                                             use_tc_tiling_on_sc=False),
    )
    return fn(emb_flat, cat2, num2, wdup, kdup, foff)


def kernel(pos_cat, pos_num, neg_cat, neg_num, emb_tables, lin_w, lin_b,
           bn_gamma, bn_beta, bn_mean, bn_var):
    emb_flat = emb_tables.reshape(_F * _V, _D)
    cat2 = jnp.concatenate([pos_cat, neg_cat], axis=1)
    num2 = jnp.concatenate([pos_num, neg_num], axis=1)
    w_emb = lin_w[0, : _F * _D].reshape(_F, _D)
    wdup = jnp.concatenate([w_emb, -w_emb], axis=0)
    knum = lin_w[0, _F * _D:] * bn_gamma * lax.rsqrt(bn_var + 1e-5)
    kdup = jnp.concatenate([knum, -knum], axis=0)
    foff = jnp.tile(jnp.arange(_F, dtype=jnp.int32) * _V, 2)
    out = _run(emb_flat, cat2, num2, wdup, kdup, foff)
    return out.reshape(_B, 1)


# speedup vs baseline: 12.3215x; 2.9355x over previous
import jax
import jax.numpy as jnp
from jax import lax
from jax.experimental import pallas as pl
from jax.experimental.pallas import tpu as pltpu
from jax.experimental.pallas import tpu_sc as plsc

_B = 4096
_F = 26
_V = 100000
_D = 32
_NC = 16

_NW = 32
_BPW = _B // _NW
_F2 = 2 * _F
_VBLK = 8192
_NVB = (_V + _VBLK - 1) // _VBLK


def _proj_body(w_ref, e_ref, s_ref):
    x = e_ref[0]
    w = w_ref[0]
    s_ref[0, 0, :] = jnp.sum(x * w, axis=0)


def _project(embT, wT):
    return pl.pallas_call(
        _proj_body,
        grid=(_F, _NVB),
        in_specs=[
            pl.BlockSpec((1, _D, 1), lambda f, vb: (f, 0, 0)),
            pl.BlockSpec((1, _D, _VBLK), lambda f, vb: (f, 0, vb)),
        ],
        out_specs=pl.BlockSpec((1, 1, _VBLK), lambda f, vb: (f, 0, vb)),
        out_shape=jax.ShapeDtypeStruct((_F, 1, _V), jnp.float32),
    )(wT, embT)


def _sc_body(s1, cat2, num2, kdup, foff, out,
             catv, numv, kv, foffv, idxv, sv, outv, sem):
    wid = lax.axis_index("s") * 2 + lax.axis_index("c")
    b0 = wid * _BPW
    iota = lax.iota(jnp.int32, 16)
    zero16 = jnp.zeros((16,), jnp.float32)

    pltpu.sync_copy(cat2.at[pl.ds(b0, _BPW), :], catv)
    pltpu.sync_copy(num2.at[pl.ds(b0, _BPW), :], numv)
    pltpu.sync_copy(kdup, kv)
    pltpu.sync_copy(foff, foffv)

    def build(fp, _):
        fpv = jnp.full((16,), fp, dtype=jnp.int32)
        fofb = plsc.load_gather(foffv, [fpv])
        for g in range(8):
            jvec = iota + (g * 16)
            catg = plsc.load_gather(catv, [jvec, fpv])
            idxv[fp, pl.ds(g * 16, 16)] = catg + fofb
        return 0

    lax.fori_loop(0, _F2, build, 0)

    descs = [
        pltpu.async_copy(s1.at[idxv.at[f]], sv.at[f], sem)
        for f in range(_F2)
    ]
    for d in descs:
        d.wait()

    for g in range(8):
        jvec = iota + (g * 16)

        def body_add(fp, a):
            return a + sv[fp, pl.ds(g * 16, 16)]

        accp = lax.fori_loop(0, _F, body_add, zero16)
        accn = lax.fori_loop(_F, _F2, body_add, zero16)
        acc = accp - accn

        def body_num(l, a):
            kb = plsc.load_gather(kv, [jnp.full((16,), l, dtype=jnp.int32)])
            nv = plsc.load_gather(numv, [jvec,
                                         jnp.full((16,), l, dtype=jnp.int32)])
            return a + kb * nv

        acc = lax.fori_loop(0, 2 * _NC, body_num, acc)
        outv[pl.ds(g * 16, 16)] = acc

    pltpu.sync_copy(outv, out.at[pl.ds(b0, _BPW)])


def _gather_reduce(s1, cat2, num2, kdup, foff):
    mesh = plsc.VectorSubcoreMesh(core_axis_name="c", subcore_axis_name="s",
                                  num_cores=2, num_subcores=16)
    fn = pl.kernel(
        _sc_body,
        out_type=jax.ShapeDtypeStruct((_B,), jnp.float32),
        mesh=mesh,
        scratch_types=[
            pltpu.VMEM((_BPW, _F2), jnp.int32),
            pltpu.VMEM((_BPW, 2 * _NC), jnp.float32),
            pltpu.VMEM((2 * _NC,), jnp.float32),
            pltpu.VMEM((_F2,), jnp.int32),
            pltpu.VMEM((_F2, _BPW), jnp.int32),
            pltpu.VMEM((_F2, _BPW), jnp.float32),
            pltpu.VMEM((_BPW,), jnp.float32),
            pltpu.SemaphoreType.DMA,
        ],
        compiler_params=pltpu.CompilerParams(needs_layout_passes=False,
                                             use_tc_tiling_on_sc=False),
    )
    return fn(s1, cat2, num2, kdup, foff)


@jax.jit
def _run(embT, wT, cat2, num2, kdup, foff):
    s = _project(embT, wT)
    return _gather_reduce(s.reshape(_F * _V), cat2, num2, kdup, foff)


def kernel(pos_cat, pos_num, neg_cat, neg_num, emb_tables, lin_w, lin_b,
           bn_gamma, bn_beta, bn_mean, bn_var):
    embT = jnp.transpose(emb_tables, (0, 2, 1))
    w_emb = lin_w[0, : _F * _D].reshape(_F, _D)
    wT = w_emb.reshape(_F, _D, 1)
    cat2 = jnp.concatenate([pos_cat, neg_cat], axis=1)
    num2 = jnp.concatenate([pos_num, neg_num], axis=1)
    knum = lin_w[0, _F * _D:] * bn_gamma * lax.rsqrt(bn_var + 1e-5)
    kdup = jnp.concatenate([knum, -knum], axis=0)
    foff = jnp.tile(jnp.arange(_F, dtype=jnp.int32) * _V, 2)
    out = _run(embT, wT, cat2, num2, kdup, foff)
    return out.reshape(_B, 1)


# speedup vs baseline: 17.3711x; 1.4098x over previous
import jax
import jax.numpy as jnp
from jax import lax
from jax.experimental import pallas as pl
from jax.experimental.pallas import tpu as pltpu
from jax.experimental.pallas import tpu_sc as plsc

_B = 4096
_F = 26
_V = 100000
_D = 32
_NC = 16

_NW = 32
_BPW = _B // _NW
_F2 = 2 * _F
_VBLK = 8192
_NVB = (_V + _VBLK - 1) // _VBLK
_VPAD = _NVB * _VBLK


def _proj_body(w_ref, e_ref, s_ref):
    x = e_ref[0]
    w = w_ref[0]
    s_ref[...] = jnp.sum(x * w, axis=0)


def _project(embT, wT):
    return pl.pallas_call(
        _proj_body,
        grid=(_F, _NVB),
        in_specs=[
            pl.BlockSpec((1, _D, 1), lambda f, vb: (f, 0, 0)),
            pl.BlockSpec((1, _D, _VBLK), lambda f, vb: (f, 0, vb)),
        ],
        out_specs=pl.BlockSpec((_VBLK,), lambda f, vb: (f * _NVB + vb,)),
        out_shape=jax.ShapeDtypeStruct((_F * _VPAD,), jnp.float32),
    )(wT, embT)


def _sc_body(s1, cat2, num2, kdup, foff, out,
             catv, numv, kv, foffv, idxv, sv, outv, sem):
    wid = lax.axis_index("s") * 2 + lax.axis_index("c")
    b0 = wid * _BPW
    iota = lax.iota(jnp.int32, 16)
    zero16 = jnp.zeros((16,), jnp.float32)

    pltpu.sync_copy(cat2.at[pl.ds(b0, _BPW), :], catv)
    pltpu.sync_copy(num2.at[pl.ds(b0, _BPW), :], numv)
    pltpu.sync_copy(kdup, kv)
    pltpu.sync_copy(foff, foffv)

    def build(fp, _):
        fpv = jnp.full((16,), fp, dtype=jnp.int32)
        fofb = plsc.load_gather(foffv, [fpv])
        for g in range(8):
            jvec = iota + (g * 16)
            catg = plsc.load_gather(catv, [jvec, fpv])
            idxv[fp, pl.ds(g * 16, 16)] = catg + fofb
        return 0

    lax.fori_loop(0, _F2, build, 0)

    descs = [
        pltpu.async_copy(s1.at[idxv.at[f]], sv.at[f], sem)
        for f in range(_F2)
    ]
    for d in descs:
        d.wait()

    for g in range(8):
        jvec = iota + (g * 16)

        def body_add(fp, a):
            return a + sv[fp, pl.ds(g * 16, 16)]

        accp = lax.fori_loop(0, _F, body_add, zero16)
        accn = lax.fori_loop(_F, _F2, body_add, zero16)
        acc = accp - accn

        def body_num(l, a):
            kb = plsc.load_gather(kv, [jnp.full((16,), l, dtype=jnp.int32)])
            nv = plsc.load_gather(numv, [jvec,
                                         jnp.full((16,), l, dtype=jnp.int32)])
            return a + kb * nv

        acc = lax.fori_loop(0, 2 * _NC, body_num, acc)
        outv[pl.ds(g * 16, 16)] = acc

    pltpu.sync_copy(outv, out.at[pl.ds(b0, _BPW)])


def _gather_reduce(s1, cat2, num2, kdup, foff):
    mesh = plsc.VectorSubcoreMesh(core_axis_name="c", subcore_axis_name="s",
                                  num_cores=2, num_subcores=16)
    fn = pl.kernel(
        _sc_body,
        out_type=jax.ShapeDtypeStruct((_B,), jnp.float32),
        mesh=mesh,
        scratch_types=[
            pltpu.VMEM((_BPW, _F2), jnp.int32),
            pltpu.VMEM((_BPW, 2 * _NC), jnp.float32),
            pltpu.VMEM((2 * _NC,), jnp.float32),
            pltpu.VMEM((_F2,), jnp.int32),
            pltpu.VMEM((_F2, _BPW), jnp.int32),
            pltpu.VMEM((_F2, _BPW), jnp.float32),
            pltpu.VMEM((_BPW,), jnp.float32),
            pltpu.SemaphoreType.DMA,
        ],
        compiler_params=pltpu.CompilerParams(needs_layout_passes=False,
                                             use_tc_tiling_on_sc=False),
    )
    return fn(s1, cat2, num2, kdup, foff)


@jax.jit
def _run(embT, wT, cat2, num2, kdup, foff):
    s = _project(embT, wT)
    return _gather_reduce(s, cat2, num2, kdup, foff)


def kernel(pos_cat, pos_num, neg_cat, neg_num, emb_tables, lin_w, lin_b,
           bn_gamma, bn_beta, bn_mean, bn_var):
    embT = jnp.transpose(emb_tables, (0, 2, 1))
    w_emb = lin_w[0, : _F * _D].reshape(_F, _D)
    wT = w_emb.reshape(_F, _D, 1)
    cat2 = jnp.concatenate([pos_cat, neg_cat], axis=1)
    num2 = jnp.concatenate([pos_num, neg_num], axis=1)
    knum = lin_w[0, _F * _D:] * bn_gamma * lax.rsqrt(bn_var + 1e-5)
    kdup = jnp.concatenate([knum, -knum], axis=0)
    foff = jnp.tile(jnp.arange(_F, dtype=jnp.int32) * _VPAD, 2)
    out = _run(embT, wT, cat2, num2, kdup, foff)
    return out.reshape(_B, 1)


# speedup vs baseline: 25.7328x; 1.4814x over previous
import jax
import jax.numpy as jnp
from jax import lax
from jax.experimental import pallas as pl
from jax.experimental.pallas import tpu as pltpu
from jax.experimental.pallas import tpu_sc as plsc

_B = 4096
_F = 26
_V = 100000
_D = 32
_NC = 16

_NW = 32
_BPW = _B // _NW
_F2 = 2 * _F
_VBLK = 32768
_NVB = (_V + _VBLK - 1) // _VBLK
_VPAD = _NVB * _VBLK


def _proj_body(w_ref, e_ref, s_ref):
    x = e_ref[0]
    w = w_ref[0]
    s_ref[...] = jnp.sum(x * w, axis=0)


def _project(embT, wT):
    return pl.pallas_call(
        _proj_body,
        grid=(_F, _NVB),
        in_specs=[
            pl.BlockSpec((1, _D, 1), lambda f, vb: (f, 0, 0)),
            pl.BlockSpec((1, _D, _VBLK), lambda f, vb: (f, 0, vb)),
        ],
        out_specs=pl.BlockSpec((_VBLK,), lambda f, vb: (f * _NVB + vb,)),
        out_shape=jax.ShapeDtypeStruct((_F * _VPAD,), jnp.float32),
    )(wT, embT)


def _sc_body(s1, cat2, num2, kdup, foff, out,
             catv, numv, kv, foffv, idxv, sv, outv, sem):
    wid = lax.axis_index("s") * 2 + lax.axis_index("c")
    b0 = wid * _BPW
    iota = lax.iota(jnp.int32, 16)
    zero16 = jnp.zeros((16,), jnp.float32)

    pltpu.sync_copy(cat2.at[pl.ds(b0, _BPW), :], catv)
    pltpu.sync_copy(num2.at[pl.ds(b0, _BPW), :], numv)
    pltpu.sync_copy(kdup, kv)
    pltpu.sync_copy(foff, foffv)

    def build(fp, _):
        fpv = jnp.full((16,), fp, dtype=jnp.int32)
        fofb = plsc.load_gather(foffv, [fpv])
        for g in range(8):
            jvec = iota + (g * 16)
            catg = plsc.load_gather(catv, [jvec, fpv])
            idxv[fp, pl.ds(g * 16, 16)] = catg + fofb
        return 0

    lax.fori_loop(0, _F2, build, 0)

    descs = [
        pltpu.async_copy(s1.at[idxv.at[f]], sv.at[f], sem)
        for f in range(_F2)
    ]
    for d in descs:
        d.wait()

    for g in range(8):
        jvec = iota + (g * 16)

        def body_add(fp, a):
            return a + sv[fp, pl.ds(g * 16, 16)]

        accp = lax.fori_loop(0, _F, body_add, zero16)
        accn = lax.fori_loop(_F, _F2, body_add, zero16)
        acc = accp - accn

        def body_num(l, a):
            kb = plsc.load_gather(kv, [jnp.full((16,), l, dtype=jnp.int32)])
            nv = plsc.load_gather(numv, [jvec,
                                         jnp.full((16,), l, dtype=jnp.int32)])
            return a + kb * nv

        acc = lax.fori_loop(0, 2 * _NC, body_num, acc)
        outv[pl.ds(g * 16, 16)] = acc

    pltpu.sync_copy(outv, out.at[pl.ds(b0, _BPW)])


def _gather_reduce(s1, cat2, num2, kdup, foff):
    mesh = plsc.VectorSubcoreMesh(core_axis_name="c", subcore_axis_name="s",
                                  num_cores=2, num_subcores=16)
    fn = pl.kernel(
        _sc_body,
        out_type=jax.ShapeDtypeStruct((_B,), jnp.float32),
        mesh=mesh,
        scratch_types=[
            pltpu.VMEM((_BPW, _F2), jnp.int32),
            pltpu.VMEM((_BPW, 2 * _NC), jnp.float32),
            pltpu.VMEM((2 * _NC,), jnp.float32),
            pltpu.VMEM((_F2,), jnp.int32),
            pltpu.VMEM((_F2, _BPW), jnp.int32),
            pltpu.VMEM((_F2, _BPW), jnp.float32),
            pltpu.VMEM((_BPW,), jnp.float32),
            pltpu.SemaphoreType.DMA,
        ],
        compiler_params=pltpu.CompilerParams(needs_layout_passes=False,
                                             use_tc_tiling_on_sc=False),
    )
    return fn(s1, cat2, num2, kdup, foff)


@jax.jit
def _run(embT, wT, cat2, num2, kdup, foff):
    s = _project(embT, wT)
    return _gather_reduce(s, cat2, num2, kdup, foff)


def kernel(pos_cat, pos_num, neg_cat, neg_num, emb_tables, lin_w, lin_b,
           bn_gamma, bn_beta, bn_mean, bn_var):
    embT = jnp.transpose(emb_tables, (0, 2, 1))
    w_emb = lin_w[0, : _F * _D].reshape(_F, _D)
    wT = w_emb.reshape(_F, _D, 1)
    cat2 = jnp.concatenate([pos_cat, neg_cat], axis=1)
    num2 = jnp.concatenate([pos_num, neg_num], axis=1)
    knum = lin_w[0, _F * _D:] * bn_gamma * lax.rsqrt(bn_var + 1e-5)
    kdup = jnp.concatenate([knum, -knum], axis=0)
    foff = jnp.tile(jnp.arange(_F, dtype=jnp.int32) * _VPAD, 2)
    out = _run(embT, wT, cat2, num2, kdup, foff)
    return out.reshape(_B, 1)


# speedup vs baseline: 30.1492x; 1.1716x over previous
import jax
import jax.numpy as jnp
from jax import lax
from jax.experimental import pallas as pl
from jax.experimental.pallas import tpu as pltpu
from jax.experimental.pallas import tpu_sc as plsc

_B = 4096
_F = 26
_V = 100000
_D = 32
_NC = 16

_NW = 32
_BPW = _B // _NW
_F2 = 2 * _F
_VBLK = 65536
_NVB = (_V + _VBLK - 1) // _VBLK
_VPAD = _NVB * _VBLK


def _proj_body(w_ref, e_ref, s_ref):
    x = e_ref[0]
    w = w_ref[0]
    s_ref[...] = jnp.sum(x * w, axis=0)


def _project(embT, wT):
    return pl.pallas_call(
        _proj_body,
        grid=(_F, _NVB),
        in_specs=[
            pl.BlockSpec((1, _D, 1), lambda f, vb: (f, 0, 0)),
            pl.BlockSpec((1, _D, _VBLK), lambda f, vb: (f, 0, vb)),
        ],
        out_specs=pl.BlockSpec((_VBLK,), lambda f, vb: (f * _NVB + vb,)),
        out_shape=jax.ShapeDtypeStruct((_F * _VPAD,), jnp.float32),
    )(wT, embT)


def _sc_body(s1, cat2, num2, kdup, foff, out,
             catv, numv, kv, foffv, idxv, sv, outv, sem):
    wid = lax.axis_index("s") * 2 + lax.axis_index("c")
    b0 = wid * _BPW
    iota = lax.iota(jnp.int32, 16)
    zero16 = jnp.zeros((16,), jnp.float32)

    pltpu.sync_copy(cat2.at[pl.ds(b0, _BPW), :], catv)
    pltpu.sync_copy(num2.at[pl.ds(b0, _BPW), :], numv)
    pltpu.sync_copy(kdup, kv)
    pltpu.sync_copy(foff, foffv)

    def build(fp, _):
        fpv = jnp.full((16,), fp, dtype=jnp.int32)
        fofb = plsc.load_gather(foffv, [fpv])
        for g in range(8):
            jvec = iota + (g * 16)
            catg = plsc.load_gather(catv, [jvec, fpv])
            idxv[fp, pl.ds(g * 16, 16)] = catg + fofb
        return 0

    lax.fori_loop(0, _F2, build, 0)

    descs = [
        pltpu.async_copy(s1.at[idxv.at[f]], sv.at[f], sem)
        for f in range(_F2)
    ]
    for d in descs:
        d.wait()

    for g in range(8):
        jvec = iota + (g * 16)

        def body_add(fp, a):
            return a + sv[fp, pl.ds(g * 16, 16)]

        accp = lax.fori_loop(0, _F, body_add, zero16)
        accn = lax.fori_loop(_F, _F2, body_add, zero16)
        acc = accp - accn

        def body_num(l, a):
            kb = plsc.load_gather(kv, [jnp.full((16,), l, dtype=jnp.int32)])
            nv = plsc.load_gather(numv, [jvec,
                                         jnp.full((16,), l, dtype=jnp.int32)])
            return a + kb * nv

        acc = lax.fori_loop(0, 2 * _NC, body_num, acc)
        outv[pl.ds(g * 16, 16)] = acc

    pltpu.sync_copy(outv, out.at[pl.ds(b0, _BPW)])


def _gather_reduce(s1, cat2, num2, kdup, foff):
    mesh = plsc.VectorSubcoreMesh(core_axis_name="c", subcore_axis_name="s",
                                  num_cores=2, num_subcores=16)
    fn = pl.kernel(
        _sc_body,
        out_type=jax.ShapeDtypeStruct((_B,), jnp.float32),
        mesh=mesh,
        scratch_types=[
            pltpu.VMEM((_BPW, _F2), jnp.int32),
            pltpu.VMEM((_BPW, 2 * _NC), jnp.float32),
            pltpu.VMEM((2 * _NC,), jnp.float32),
            pltpu.VMEM((_F2,), jnp.int32),
            pltpu.VMEM((_F2, _BPW), jnp.int32),
            pltpu.VMEM((_F2, _BPW), jnp.float32),
            pltpu.VMEM((_BPW,), jnp.float32),
            pltpu.SemaphoreType.DMA,
        ],
        compiler_params=pltpu.CompilerParams(needs_layout_passes=False,
                                             use_tc_tiling_on_sc=False),
    )
    return fn(s1, cat2, num2, kdup, foff)


@jax.jit
def _run(embT, wT, cat2, num2, kdup, foff):
    s = _project(embT, wT)
    return _gather_reduce(s, cat2, num2, kdup, foff)


def kernel(pos_cat, pos_num, neg_cat, neg_num, emb_tables, lin_w, lin_b,
           bn_gamma, bn_beta, bn_mean, bn_var):
    embT = jnp.transpose(emb_tables, (0, 2, 1))
    w_emb = lin_w[0, : _F * _D].reshape(_F, _D)
    wT = w_emb.reshape(_F, _D, 1)
    cat2 = jnp.concatenate([pos_cat, neg_cat], axis=1)
    num2 = jnp.concatenate([pos_num, neg_num], axis=1)
    knum = lin_w[0, _F * _D:] * bn_gamma * lax.rsqrt(bn_var + 1e-5)
    kdup = jnp.concatenate([knum, -knum], axis=0)
    foff = jnp.tile(jnp.arange(_F, dtype=jnp.int32) * _VPAD, 2)
    out = _run(embT, wT, cat2, num2, kdup, foff)
    return out.reshape(_B, 1)


# speedup vs baseline: 35.5886x; 1.1804x over previous
import jax
import jax.numpy as jnp
from jax import lax
from jax.experimental import pallas as pl
from jax.experimental.pallas import tpu as pltpu
from jax.experimental.pallas import tpu_sc as plsc

_B = 4096
_F = 26
_V = 100000
_D = 32
_NC = 16

_NW = 32
_BPW = _B // _NW
_F2 = 2 * _F
_VBLK = 102400
_NVB = (_V + _VBLK - 1) // _VBLK
_VPAD = _NVB * _VBLK


def _proj_body(w_ref, e_ref, s_ref):
    x = e_ref[0]
    w = w_ref[0]
    s_ref[...] = jnp.sum(x * w, axis=0)


def _project(embT, wT):
    return pl.pallas_call(
        _proj_body,
        grid=(_F, _NVB),
        in_specs=[
            pl.BlockSpec((1, _D, 1), lambda f, vb: (f, 0, 0)),
            pl.BlockSpec((1, _D, _VBLK), lambda f, vb: (f, 0, vb)),
        ],
        out_specs=pl.BlockSpec((_VBLK,), lambda f, vb: (f * _NVB + vb,)),
        out_shape=jax.ShapeDtypeStruct((_F * _VPAD,), jnp.float32),
    )(wT, embT)


def _sc_body(s1, cat2, num2, kdup, foff, out,
             catv, numv, kv, foffv, idxv, sv, outv, sem):
    wid = lax.axis_index("s") * 2 + lax.axis_index("c")
    b0 = wid * _BPW
    iota = lax.iota(jnp.int32, 16)
    zero16 = jnp.zeros((16,), jnp.float32)

    pltpu.sync_copy(cat2.at[pl.ds(b0, _BPW), :], catv)
    pltpu.sync_copy(num2.at[pl.ds(b0, _BPW), :], numv)
    pltpu.sync_copy(kdup, kv)
    pltpu.sync_copy(foff, foffv)

    def build(fp, _):
        fpv = jnp.full((16,), fp, dtype=jnp.int32)
        fofb = plsc.load_gather(foffv, [fpv])
        for g in range(8):
            jvec = iota + (g * 16)
            catg = plsc.load_gather(catv, [jvec, fpv])
            idxv[fp, pl.ds(g * 16, 16)] = catg + fofb
        return 0

    lax.fori_loop(0, _F2, build, 0)

    descs = [
        pltpu.async_copy(s1.at[idxv.at[f]], sv.at[f], sem)
        for f in range(_F2)
    ]
    for d in descs:
        d.wait()

    for g in range(8):
        jvec = iota + (g * 16)

        def body_add(fp, a):
            return a + sv[fp, pl.ds(g * 16, 16)]

        accp = lax.fori_loop(0, _F, body_add, zero16)
        accn = lax.fori_loop(_F, _F2, body_add, zero16)
        acc = accp - accn

        def body_num(l, a):
            kb = plsc.load_gather(kv, [jnp.full((16,), l, dtype=jnp.int32)])
            nv = plsc.load_gather(numv, [jvec,
                                         jnp.full((16,), l, dtype=jnp.int32)])
            return a + kb * nv

        acc = lax.fori_loop(0, 2 * _NC, body_num, acc)
        outv[pl.ds(g * 16, 16)] = acc

    pltpu.sync_copy(outv, out.at[pl.ds(b0, _BPW)])


def _gather_reduce(s1, cat2, num2, kdup, foff):
    mesh = plsc.VectorSubcoreMesh(core_axis_name="c", subcore_axis_name="s",
                                  num_cores=2, num_subcores=16)
    fn = pl.kernel(
        _sc_body,
        out_type=jax.ShapeDtypeStruct((_B,), jnp.float32),
        mesh=mesh,
        scratch_types=[
            pltpu.VMEM((_BPW, _F2), jnp.int32),
            pltpu.VMEM((_BPW, 2 * _NC), jnp.float32),
            pltpu.VMEM((2 * _NC,), jnp.float32),
            pltpu.VMEM((_F2,), jnp.int32),
            pltpu.VMEM((_F2, _BPW), jnp.int32),
            pltpu.VMEM((_F2, _BPW), jnp.float32),
            pltpu.VMEM((_BPW,), jnp.float32),
            pltpu.SemaphoreType.DMA,
        ],
        compiler_params=pltpu.CompilerParams(needs_layout_passes=False,
                                             use_tc_tiling_on_sc=False),
    )
    return fn(s1, cat2, num2, kdup, foff)


@jax.jit
def _run(embT, wT, cat2, num2, kdup, foff):
    s = _project(embT, wT)
    return _gather_reduce(s, cat2, num2, kdup, foff)


def kernel(pos_cat, pos_num, neg_cat, neg_num, emb_tables, lin_w, lin_b,
           bn_gamma, bn_beta, bn_mean, bn_var):
    embT = jnp.transpose(emb_tables, (0, 2, 1))
    w_emb = lin_w[0, : _F * _D].reshape(_F, _D)
    wT = w_emb.reshape(_F, _D, 1)
    cat2 = jnp.concatenate([pos_cat, neg_cat], axis=1)
    num2 = jnp.concatenate([pos_num, neg_num], axis=1)
    knum = lin_w[0, _F * _D:] * bn_gamma * lax.rsqrt(bn_var + 1e-5)
    kdup = jnp.concatenate([knum, -knum], axis=0)
    foff = jnp.tile(jnp.arange(_F, dtype=jnp.int32) * _VPAD, 2)
    out = _run(embT, wT, cat2, num2, kdup, foff)
    return out.reshape(_B, 1)
